# baseline (speedup 1.0000x reference)
import jax
import jax.numpy as jnp
from jax import lax
from jax.experimental import pallas as pl


def _matmul_body(u_ref, it_ref, o_ref):
  acc = lax.dot_general(
      u_ref[...], it_ref[...],
      dimension_numbers=(((1,), (1,)), ((), ())),
      preferred_element_type=jnp.float32,
  )
  o_ref[...] = acc.astype(jnp.bfloat16)


def _tc_scores(emb, batch, dim):
  bu = 1024
  bi = 4096
  grid = (batch // bu, batch // bi)
  item_block_off = batch // bi

  out = pl.pallas_call(
      _matmul_body,
      grid=grid,
      in_specs=[
          pl.BlockSpec((bu, dim), lambda i, j: (i, 0)),
          pl.BlockSpec((bi, dim), lambda i, j: (j + item_block_off, 0)),
      ],
      out_specs=pl.BlockSpec((bu, bi), lambda i, j: (i, j)),
      out_shape=jax.ShapeDtypeStruct((batch, batch), jnp.bfloat16),
  )(emb, emb)
  return out.astype(jnp.float32)


@jax.jit
def kernel(id_embedding, user_tensor, item_tensor):
  batch = user_tensor.shape[0]
  dim = id_embedding.shape[1]
  idx = jnp.concatenate(
      [user_tensor.astype(jnp.int32), item_tensor.astype(jnp.int32)])
  emb = jnp.take(id_embedding, idx, axis=0)
  return _tc_scores(emb, batch, dim)
